# baseline (speedup 1.0000x reference)
import functools

import jax
import jax.numpy as jnp
from jax import lax
from jax.experimental import pallas as pl
from jax.experimental.pallas import tpu as pltpu
from jax.experimental.pallas import tpu_sc as plsc

_VOCAB = 100000
_D = 64
_B = 4096
_H = 200
_C = 5
_DP = 16

_NC = 2
_NS = 16
_NW = _NC * _NS
_BPW = _B // _NW

_VBLK = 4096
_NSLOT = 4


def _proj_body(tt_ref, w_ref, o_ref):
    prod = lax.dot_general(tt_ref[...], w_ref[...],
                           dimension_numbers=(((0,), (0,)), ((), ())),
                           preferred_element_type=jnp.float32)
    o_ref[:, : _DP] = prod


def _project(table_t, wp):
    return pl.pallas_call(
        _proj_body,
        grid=(-(-_VOCAB // _VBLK),),
        in_specs=[
            pl.BlockSpec((_D, _VBLK), lambda i: (0, i)),
            pl.BlockSpec((_D, _DP), lambda i: (0, 0)),
        ],
        out_specs=pl.BlockSpec((_VBLK, 128), lambda i: (i, 0)),
        out_shape=jax.ShapeDtypeStruct((_VOCAB, 128), jnp.float32),
    )(table_t, wp)


def _make_pool():
    mesh = plsc.VectorSubcoreMesh(core_axis_name="c", subcore_axis_name="s")

    @functools.partial(
        pl.kernel,
        mesh=mesh,
        out_type=jax.ShapeDtypeStruct((_B, _DP), jnp.float32),
        scratch_types=[
            pltpu.VMEM((_H // 8, 8, _BPW), jnp.int32),
            pltpu.VMEM((_NSLOT, _BPW, _DP), jnp.float32),
            pltpu.VMEM((_BPW, _DP), jnp.float32),
            pltpu.VMEM((_DP,), jnp.float32),
            [pltpu.SemaphoreType.DMA] * _NSLOT,
        ],
        compiler_params=pltpu.CompilerParams(use_tc_tiling_on_sc=False),
    )
    def pool(x4_hbm, tw_hbm, bias_hbm, out_hbm, idx_v, rows_v, acc_v, bias_v,
             sems):
        wid = lax.axis_index("s") * _NC + lax.axis_index("c")
        base = wid * _BPW
        pltpu.sync_copy(x4_hbm.at[:, wid], idx_v)
        pltpu.sync_copy(bias_hbm, bias_v)
        bias = bias_v[...]
        for b in range(_BPW):
            acc_v[b] = bias

        def scale_row(j, carry):
            for q in range(_BPW // 16):
                sl = pl.ds(16 * q, 16)
                idx_v[j >> 3, j & 7, sl] = idx_v[j >> 3, j & 7, sl] * 8
            return carry

        lax.fori_loop(0, _H, scale_row, 0)

        def issue(j, s):
            pltpu.async_copy(tw_hbm.at[idx_v.at[j >> 3, j & 7]], rows_v.at[s],
                             sems[s])

        def drain(s):
            pltpu.make_async_copy(tw_hbm.at[idx_v.at[0, 0]], rows_v.at[s],
                                  sems[s]).wait()

        for s in range(_NSLOT):
            issue(s, s)

        def group(g, carry):
            for s in range(_NSLOT):
                j = _NSLOT * g + s
                drain(s)
                for b in range(_BPW):
                    plsc.addupdate(acc_v.at[b], rows_v[s, b])

                @pl.when(j + _NSLOT < _H)
                def _():
                    issue(j + _NSLOT, s)
            return carry

        lax.fori_loop(0, _H // _NSLOT, group, 0)
        pltpu.sync_copy(acc_v, out_hbm.at[pl.ds(base, _BPW)])

    return pool


_pool_call = _make_pool()


def kernel(x, table, W, b):
    x = x.astype(jnp.int32)
    wp = jnp.pad(W, ((0, 0), (0, _DP - _C)))
    bp = jnp.pad(b, (0, _DP - _C))
    tw_pad = _project(table.T, wp)
    tw = tw_pad.reshape(8 * _VOCAB, _DP)
    x4 = x.T.reshape(_H // 8, 8, _B // _BPW, _BPW).transpose(0, 2, 1, 3)
    out16 = _pool_call(x4, tw, bp)
    return out16[:, :_C]
